# baseline (speedup 1.0000x reference)
import functools

import jax
import jax.numpy as jnp
from jax import lax
from jax.experimental import pallas as pl
from jax.experimental.pallas import tpu as pltpu
from jax.experimental.pallas import tpu_sc as plsc

H = 128
_NC, _NS = 2, 16
_NW = _NC * _NS
_L = 128
_EPS = 1e-5


def _cdiv(a, b):
    return (a + b - 1) // b


def _sc_gather2(tab_s, tab_d, idxr, idxc):
    C, L = idxr.shape
    D = tab_s.shape[1]
    n_iter = C // _NW
    assert C == n_iter * _NW and n_iter % 2 == 0
    mesh = plsc.VectorSubcoreMesh(core_axis_name="c", subcore_axis_name="s",
                                  num_cores=_NC, num_subcores=_NS)
    out = jax.ShapeDtypeStruct((C * L, D), jnp.float32)
    rows_t = pltpu.VMEM((L, D), jnp.float32)

    @functools.partial(
        pl.kernel,
        out_type=(out, out),
        mesh=mesh,
        scratch_types=[
            pltpu.VMEM((n_iter, L), jnp.int32),
            pltpu.VMEM((n_iter, L), jnp.int32),
            rows_t, rows_t, rows_t, rows_t,
            pltpu.SemaphoreType.DMA, pltpu.SemaphoreType.DMA,
            pltpu.SemaphoreType.DMA, pltpu.SemaphoreType.DMA,
        ],
    )
    def gk(ts_hbm, td_hbm, idxr_hbm, idxc_hbm, gs_hbm, gd_hbm,
           ir_v, ic_v, rs0, rd0, rs1, rd1, sg0, sg1, so0, so1):
        wid = lax.axis_index("s") * _NC + lax.axis_index("c")
        j0 = wid * n_iter
        rs = (rs0, rs1)
        rd = (rd0, rd1)
        sg = (sg0, sg1)
        so = (so0, so1)

        pltpu.sync_copy(idxr_hbm.at[pl.ds(j0, n_iter)], ir_v)
        pltpu.sync_copy(idxc_hbm.at[pl.ds(j0, n_iter)], ic_v)
        pltpu.async_copy(ts_hbm.at[ir_v.at[0]], rs0, sg0)
        pltpu.async_copy(td_hbm.at[ic_v.at[0]], rd0, sg0)

        def slot(i, b, t):
            nb = 1 - b
            pltpu.make_async_copy(ts_hbm.at[ir_v.at[0]], rs[b], sg[b]).wait()
            pltpu.make_async_copy(td_hbm.at[ic_v.at[0]], rd[b], sg[b]).wait()
            pltpu.async_copy(rs[b], gs_hbm.at[pl.ds((j0 + i) * L, L)], so[b])
            pltpu.async_copy(rd[b], gd_hbm.at[pl.ds((j0 + i) * L, L)], so[b])

            def prefetch():
                def wait_out_nb():
                    pltpu.make_async_copy(rs[nb], gs_hbm.at[pl.ds(0, L)], so[nb]).wait()
                    pltpu.make_async_copy(rd[nb], gd_hbm.at[pl.ds(0, L)], so[nb]).wait()

                if b == 0:
                    pl.when(t > 0)(wait_out_nb)
                else:
                    wait_out_nb()
                pltpu.async_copy(ts_hbm.at[ir_v.at[i + 1]], rs[nb], sg[nb])
                pltpu.async_copy(td_hbm.at[ic_v.at[i + 1]], rd[nb], sg[nb])

            if b == 0:
                prefetch()
            else:
                pl.when(t < n_iter // 2 - 1)(prefetch)

        def outer(t, carry):
            slot(2 * t, 0, t)
            slot(2 * t + 1, 1, t)
            return carry

        lax.fori_loop(0, n_iter // 2, outer, 0)
        for b in (0, 1):
            pltpu.make_async_copy(rs[b], gs_hbm.at[pl.ds(0, L)], so[b]).wait()
            pltpu.make_async_copy(rd[b], gd_hbm.at[pl.ds(0, L)], so[b]).wait()

    return gk(tab_s, tab_d, idxr, idxc)


def _sc_scatter(vals, idx2d, zeros_nd, n_real):
    C, L = idx2d.shape
    Nn, D = zeros_nd.shape
    rpt = (n_real // _NS) // 8 * 8
    tail = n_real - rpt * _NS
    n_iter = C // _NW
    assert C == n_iter * _NW and n_iter % 2 == 0
    mesh = plsc.VectorSubcoreMesh(core_axis_name="c", subcore_axis_name="s",
                                  num_cores=_NC, num_subcores=_NS)
    vals_t = pltpu.VMEM((L, D), jnp.float32)

    @functools.partial(
        pl.kernel,
        out_type=jax.ShapeDtypeStruct((_NC * n_real, D), jnp.float32),
        mesh=mesh,
        scratch_types=[
            pltpu.VMEM((n_iter, L), jnp.int32),
            vals_t, vals_t,
            pltpu.VMEM_SHARED((Nn, D), jnp.float32),
            pltpu.SemaphoreType.DMA, pltpu.SemaphoreType.DMA,
            pltpu.SemaphoreType.DMA,
        ],
    )
    def sk(vals_hbm, idx_hbm, zeros_hbm, out_hbm,
           idx_v, v0, v1, acc, sv0, sv1, ss):
        cid = lax.axis_index("c")
        sid = lax.axis_index("s")
        wid = sid * _NC + cid
        j0 = wid * n_iter
        vb = (v0, v1)
        sv = (sv0, sv1)
        r0 = sid * rpt

        pltpu.sync_copy(zeros_hbm.at[pl.ds(r0, rpt)], acc.at[pl.ds(r0, rpt)])

        @pl.when(sid == _NS - 1)
        def _():
            pltpu.sync_copy(zeros_hbm.at[pl.ds(rpt * _NS, Nn - rpt * _NS)],
                            acc.at[pl.ds(rpt * _NS, Nn - rpt * _NS)])

        pltpu.sync_copy(idx_hbm.at[pl.ds(j0, n_iter)], idx_v)
        pltpu.async_copy(vals_hbm.at[pl.ds(j0 * L, L)], v0, sv0)
        plsc.subcore_barrier()

        def slot(i, b, t):
            nb = 1 - b

            def wait_scatter_prev():
                pltpu.make_async_copy(v0, acc.at[idx_v.at[0]], ss).wait()

            def load_next():
                pltpu.async_copy(vals_hbm.at[pl.ds((j0 + i + 1) * L, L)],
                                 vb[nb], sv[nb])

            if b == 0:
                pl.when(t > 0)(wait_scatter_prev)
                load_next()
            else:
                wait_scatter_prev()
                pl.when(t < n_iter // 2 - 1)(load_next)
            pltpu.make_async_copy(vals_hbm.at[pl.ds(0, L)], vb[b], sv[b]).wait()
            pltpu.async_copy(vb[b], acc.at[idx_v.at[i]], ss, add=True)

        def outer(t, carry):
            slot(2 * t, 0, t)
            slot(2 * t + 1, 1, t)
            return carry

        lax.fori_loop(0, n_iter // 2, outer, 0)
        pltpu.make_async_copy(v0, acc.at[idx_v.at[0]], ss).wait()
        plsc.subcore_barrier()
        pltpu.sync_copy(acc.at[pl.ds(r0, rpt)],
                        out_hbm.at[pl.ds(cid * n_real + r0, rpt)])
        if tail:
            @pl.when(sid == _NS - 1)
            def _():
                pltpu.sync_copy(acc.at[pl.ds(rpt * _NS, tail)],
                                out_hbm.at[pl.ds(cid * n_real + rpt * _NS, tail)])

    return sk(vals, idx2d, zeros_nd)


def _ln(x, g, b):
    mu = jnp.mean(x, axis=-1, keepdims=True)
    xc = x - mu
    var = jnp.mean(xc * xc, axis=-1, keepdims=True)
    return xc * lax.rsqrt(var + _EPS) * g + b


def _full(shape):
    return pl.BlockSpec(shape, lambda i: (0, 0))


def _rows(blk, d):
    return pl.BlockSpec((blk, d), lambda i: (i, 0))


def _tc_vencode(vin, w1, b1, w2, b2, g, b, w1s, w1d, blk):
    Nn = vin.shape[0]

    def body(x_ref, w1_ref, b1_ref, w2_ref, b2_ref, g_ref, b_ref,
             ws_ref, wd_ref, v_ref, ps_ref, pd_ref):
        h = jnp.maximum(jnp.dot(x_ref[...], w1_ref[...],
                                preferred_element_type=jnp.float32) + b1_ref[...], 0.0)
        v = jnp.dot(h, w2_ref[...], preferred_element_type=jnp.float32) + b2_ref[...]
        v = _ln(v, g_ref[...], b_ref[...])
        v_ref[...] = v
        ps_ref[...] = jnp.dot(v, ws_ref[...], preferred_element_type=jnp.float32)
        pd_ref[...] = jnp.dot(v, wd_ref[...], preferred_element_type=jnp.float32)

    out = jax.ShapeDtypeStruct((Nn, H), jnp.float32)
    return pl.pallas_call(
        body,
        grid=(Nn // blk,),
        in_specs=[_rows(blk, vin.shape[1]), _full(w1.shape), _full((1, H)),
                  _full((H, H)), _full((1, H)), _full((1, H)), _full((1, H)),
                  _full((H, H)), _full((H, H))],
        out_specs=[_rows(blk, H)] * 3,
        out_shape=[out, out, out],
    )(vin, w1, b1, w2, b2, g, b, w1s, w1d)


def _tc_eencode(grow, gcol, w1, b1, w2, b2, g, b, blk):
    E = grow.shape[0]

    def body(gr_ref, gc_ref, w1_ref, b1_ref, w2_ref, b2_ref, g_ref, b_ref, o_ref):
        d = gc_ref[...] - gr_ref[...]
        w1 = w1_ref[...]
        sqe = jnp.sum(d[:, 0:3] * d[:, 0:3], axis=-1, keepdims=True)
        ne = jnp.where(sqe > 0, jnp.sqrt(jnp.where(sqe > 0, sqe, 1.0)), 0.0)
        sqm = jnp.sum(d[:, 3:6] * d[:, 3:6], axis=-1, keepdims=True)
        nm = jnp.where(sqm > 0, jnp.sqrt(jnp.where(sqm > 0, sqm, 1.0)), 0.0)
        h = jnp.broadcast_to(b1_ref[...], (blk, H))
        for k in range(3):
            h = h + d[:, k:k + 1] * w1[k:k + 1, :]
        h = h + ne * w1[3:4, :]
        for k in range(3):
            h = h + d[:, 3 + k:4 + k] * w1[4 + k:5 + k, :]
        h = h + nm * w1[7:8, :]
        h = jnp.maximum(h, 0.0)
        e = jnp.dot(h, w2_ref[...], preferred_element_type=jnp.float32) + b2_ref[...]
        o_ref[...] = _ln(e, g_ref[...], b_ref[...])

    return pl.pallas_call(
        body,
        grid=(E // blk,),
        in_specs=[_rows(blk, grow.shape[1]), _rows(blk, grow.shape[1]),
                  _full((8, H)), _full((1, H)), _full((H, H)), _full((1, H)),
                  _full((1, H)), _full((1, H))],
        out_specs=_rows(blk, H),
        out_shape=jax.ShapeDtypeStruct((E, H), jnp.float32),
    )(grow, gcol, w1, b1, w2, b2, g, b)


def _tc_edge(gs, gd, ea, w1e, b1, w2, b2, g, b, blk):
    E = ea.shape[0]

    def body(gs_ref, gd_ref, ea_ref, w1_ref, b1_ref, w2_ref, b2_ref,
             g_ref, b_ref, o_ref):
        ea_v = ea_ref[...]
        h = jnp.maximum(
            gs_ref[...] + gd_ref[...] + b1_ref[...]
            + jnp.dot(ea_v, w1_ref[...], preferred_element_type=jnp.float32), 0.0)
        e = jnp.dot(h, w2_ref[...], preferred_element_type=jnp.float32) \
            + b2_ref[...] + ea_v
        o_ref[...] = _ln(e, g_ref[...], b_ref[...])

    return pl.pallas_call(
        body,
        grid=(E // blk,),
        in_specs=[_rows(blk, H)] * 3
        + [_full((H, H)), _full((1, H)), _full((H, H)), _full((1, H)),
           _full((1, H)), _full((1, H))],
        out_specs=_rows(blk, H),
        out_shape=jax.ShapeDtypeStruct((E, H), jnp.float32),
    )(gs, gd, ea, w1e, b1, w2, b2, g, b)


def _tc_node(v, a0, a1, w1v, w1a, b1, w2, b2, g, b, w1s, w1d, blk):
    Nn = v.shape[0]

    def body(v_ref, a0_ref, a1_ref, w1v_ref, w1a_ref, b1_ref, w2_ref, b2_ref,
             g_ref, b_ref, ws_ref, wd_ref, vn_ref, ps_ref, pd_ref):
        v_v = v_ref[...]
        agg = a0_ref[...] + a1_ref[...]
        h = jnp.maximum(
            jnp.dot(v_v, w1v_ref[...], preferred_element_type=jnp.float32)
            + jnp.dot(agg, w1a_ref[...], preferred_element_type=jnp.float32)
            + b1_ref[...], 0.0)
        x = jnp.dot(h, w2_ref[...], preferred_element_type=jnp.float32) \
            + b2_ref[...] + v_v
        vn = _ln(x, g_ref[...], b_ref[...])
        vn_ref[...] = vn
        ps_ref[...] = jnp.dot(vn, ws_ref[...], preferred_element_type=jnp.float32)
        pd_ref[...] = jnp.dot(vn, wd_ref[...], preferred_element_type=jnp.float32)

    out = jax.ShapeDtypeStruct((Nn, H), jnp.float32)
    return pl.pallas_call(
        body,
        grid=(Nn // blk,),
        in_specs=[_rows(blk, H)] * 3
        + [_full((H, H)), _full((H, H)), _full((1, H)), _full((H, H)),
           _full((1, H)), _full((1, H)), _full((1, H)), _full((H, H)),
           _full((H, H))],
        out_specs=[_rows(blk, H)] * 3,
        out_shape=[out, out, out],
    )(v, a0, a1, w1v, w1a, b1, w2, b2, g, b, w1s, w1d)


def _tc_decode(v, w1, b1, w2p, b2p, blk):
    Nn = v.shape[0]

    def body(v_ref, w1_ref, b1_ref, w2_ref, b2_ref, o_ref):
        h = jnp.maximum(jnp.dot(v_ref[...], w1_ref[...],
                                preferred_element_type=jnp.float32) + b1_ref[...], 0.0)
        o_ref[...] = jnp.dot(h, w2_ref[...],
                             preferred_element_type=jnp.float32) + b2_ref[...]

    return pl.pallas_call(
        body,
        grid=(Nn // blk,),
        in_specs=[_rows(blk, H), _full((H, H)), _full((1, H)), _full((H, H)),
                  _full((1, H))],
        out_specs=_rows(blk, H),
        out_shape=jax.ShapeDtypeStruct((Nn, H), jnp.float32),
    )(v, w1, b1, w2p, b2p)


def kernel(world_coords, vertex_features, edge_index, static_nodes, mesh_coords,
           venc_w1, venc_b1, venc_w2, venc_b2,
           eenc_w1, eenc_b1, eenc_w2, eenc_b2,
           ln_g, ln_b,
           em_w1, em_b1, em_w2, em_b2, em_ln_g, em_ln_b,
           nm_w1, nm_b1, nm_w2, nm_b2, nm_ln_g, nm_ln_b,
           dec_w1, dec_b1, dec_w2, dec_b2):
    f32 = jnp.float32
    Nn = world_coords.shape[1]
    E = edge_index.shape[2]
    OUT = dec_w2.shape[1]
    nblk = 2000 if Nn % 2000 == 0 else Nn

    def r1(x):
        return x.reshape(1, -1)

    row0 = edge_index[0, 0]
    col0 = edge_index[0, 1]
    perm = jnp.argsort(row0)
    row0 = row0[perm]
    col0 = col0[perm]

    n_it = _cdiv(_cdiv(E, _L), _NW)
    n_it += n_it % 2
    C = n_it * _NW
    Epad = C * _L
    padg = jnp.zeros((Epad - E,), jnp.int32)
    rowg = jnp.concatenate([row0, padg]).reshape(C, _L)
    colg = jnp.concatenate([col0, padg]).reshape(C, _L)
    rows_idx = jnp.concatenate(
        [row0, jnp.full((Epad - E,), Nn, jnp.int32)]).reshape(C, _L)
    eblk = 2048 if Epad % 2048 == 0 else _L

    ct = jnp.concatenate(
        [world_coords[0], mesh_coords, jnp.zeros((Nn, H - 6), f32)], axis=1)
    grow, gcol = _sc_gather2(ct, ct, rowg, colg)
    ea = _tc_eencode(grow, gcol, eenc_w1, r1(eenc_b1), eenc_w2, r1(eenc_b2),
                     r1(ln_g), r1(ln_b), eblk)

    static_oh = jax.nn.one_hot(static_nodes, 2, dtype=f32)
    vin = jnp.concatenate(
        [static_oh, vertex_features[0], jnp.zeros((Nn, 4), f32)], axis=1)
    venc_w1p = jnp.concatenate([venc_w1, jnp.zeros((4, H), f32)], axis=0)
    em_w1s, em_w1d, em_w1e = em_w1[:H], em_w1[H:2 * H], em_w1[2 * H:]
    v, ps, pd = _tc_vencode(vin, venc_w1p, r1(venc_b1), venc_w2, r1(venc_b2),
                            r1(ln_g), r1(ln_b), em_w1s, em_w1d, nblk)

    zn = jnp.zeros((Nn + 16, H), f32)
    for _ in range(15):
        gs, gd = _sc_gather2(ps, pd, rowg, colg)
        ea = _tc_edge(gs, gd, ea, em_w1e, r1(em_b1), em_w2, r1(em_b2),
                      r1(em_ln_g), r1(em_ln_b), eblk)
        agg = _sc_scatter(ea, rows_idx, zn, Nn)
        v, ps, pd = _tc_node(v, agg[:Nn], agg[Nn:], nm_w1[:H], nm_w1[H:],
                             r1(nm_b1), nm_w2, r1(nm_b2),
                             r1(nm_ln_g), r1(nm_ln_b), em_w1s, em_w1d, nblk)

    dec_w2p = jnp.concatenate([dec_w2, jnp.zeros((H, H - OUT), f32)], axis=1)
    dec_b2p = jnp.concatenate([dec_b2, jnp.zeros((H - OUT,), f32)]).reshape(1, H)
    out = _tc_decode(v, dec_w1, r1(dec_b1), dec_w2p, dec_b2p, nblk)
    return out[:, :OUT].reshape(1, Nn, OUT)


# speedup vs baseline: 1.2092x; 1.2092x over previous
import functools

import jax
import jax.numpy as jnp
from jax import lax
from jax.experimental import pallas as pl
from jax.experimental.pallas import tpu as pltpu
from jax.experimental.pallas import tpu_sc as plsc

H = 128
_NC, _NS = 2, 16
_NW = _NC * _NS
_L = 128
_EPS = 1e-5


def _cdiv(a, b):
    return (a + b - 1) // b


def _sc_gather2(tab_s, tab_d, idxr, idxc):
    C, L = idxr.shape
    D = tab_s.shape[1]
    n_iter = C // _NW
    assert C == n_iter * _NW and n_iter % 2 == 0
    mesh = plsc.VectorSubcoreMesh(core_axis_name="c", subcore_axis_name="s",
                                  num_cores=_NC, num_subcores=_NS)
    out = jax.ShapeDtypeStruct((C * L, D), jnp.float32)
    rows_t = pltpu.VMEM((L, D), jnp.float32)

    @functools.partial(
        pl.kernel,
        out_type=(out, out),
        mesh=mesh,
        scratch_types=[
            pltpu.VMEM((n_iter, L), jnp.int32),
            pltpu.VMEM((n_iter, L), jnp.int32),
            rows_t, rows_t, rows_t, rows_t,
            pltpu.SemaphoreType.DMA, pltpu.SemaphoreType.DMA,
            pltpu.SemaphoreType.DMA, pltpu.SemaphoreType.DMA,
        ],
    )
    def gk(ts_hbm, td_hbm, idxr_hbm, idxc_hbm, gs_hbm, gd_hbm,
           ir_v, ic_v, rs0, rd0, rs1, rd1, sg0, sg1, so0, so1):
        wid = lax.axis_index("s") * _NC + lax.axis_index("c")
        j0 = wid * n_iter
        rs = (rs0, rs1)
        rd = (rd0, rd1)
        sg = (sg0, sg1)
        so = (so0, so1)

        pltpu.sync_copy(idxr_hbm.at[pl.ds(j0, n_iter)], ir_v)
        pltpu.sync_copy(idxc_hbm.at[pl.ds(j0, n_iter)], ic_v)
        pltpu.async_copy(ts_hbm.at[ir_v.at[0]], rs0, sg0)
        pltpu.async_copy(td_hbm.at[ic_v.at[0]], rd0, sg0)

        def slot(i, b, t):
            nb = 1 - b
            pltpu.make_async_copy(ts_hbm.at[ir_v.at[0]], rs[b], sg[b]).wait()
            pltpu.make_async_copy(td_hbm.at[ic_v.at[0]], rd[b], sg[b]).wait()
            pltpu.async_copy(rs[b], gs_hbm.at[pl.ds((j0 + i) * L, L)], so[b])
            pltpu.async_copy(rd[b], gd_hbm.at[pl.ds((j0 + i) * L, L)], so[b])

            def prefetch():
                def wait_out_nb():
                    pltpu.make_async_copy(rs[nb], gs_hbm.at[pl.ds(0, L)], so[nb]).wait()
                    pltpu.make_async_copy(rd[nb], gd_hbm.at[pl.ds(0, L)], so[nb]).wait()

                if b == 0:
                    pl.when(t > 0)(wait_out_nb)
                else:
                    wait_out_nb()
                pltpu.async_copy(ts_hbm.at[ir_v.at[i + 1]], rs[nb], sg[nb])
                pltpu.async_copy(td_hbm.at[ic_v.at[i + 1]], rd[nb], sg[nb])

            if b == 0:
                prefetch()
            else:
                pl.when(t < n_iter // 2 - 1)(prefetch)

        def outer(t, carry):
            slot(2 * t, 0, t)
            slot(2 * t + 1, 1, t)
            return carry

        lax.fori_loop(0, n_iter // 2, outer, 0)
        for b in (0, 1):
            pltpu.make_async_copy(rs[b], gs_hbm.at[pl.ds(0, L)], so[b]).wait()
            pltpu.make_async_copy(rd[b], gd_hbm.at[pl.ds(0, L)], so[b]).wait()

    return gk(tab_s, tab_d, idxr, idxc)


def _sc_scatter(vals, idx2d, zeros_nd, n_real):
    C, L = idx2d.shape
    Nn, D = zeros_nd.shape
    rpt = (n_real // _NS) // 8 * 8
    tail = n_real - rpt * _NS
    n_iter = C // _NW
    assert C == n_iter * _NW and n_iter % 2 == 0
    mesh = plsc.VectorSubcoreMesh(core_axis_name="c", subcore_axis_name="s",
                                  num_cores=_NC, num_subcores=_NS)
    vals_t = pltpu.VMEM((L, D), jnp.float32)

    @functools.partial(
        pl.kernel,
        out_type=jax.ShapeDtypeStruct((_NC * n_real, D), jnp.float32),
        mesh=mesh,
        scratch_types=[
            pltpu.VMEM((n_iter, L), jnp.int32),
            vals_t, vals_t,
            pltpu.VMEM_SHARED((Nn, D), jnp.float32),
            pltpu.SemaphoreType.DMA, pltpu.SemaphoreType.DMA,
            pltpu.SemaphoreType.DMA,
        ],
    )
    def sk(vals_hbm, idx_hbm, zeros_hbm, out_hbm,
           idx_v, v0, v1, acc, sv0, sv1, ss):
        cid = lax.axis_index("c")
        sid = lax.axis_index("s")
        wid = sid * _NC + cid
        j0 = wid * n_iter
        vb = (v0, v1)
        sv = (sv0, sv1)
        r0 = sid * rpt

        pltpu.sync_copy(zeros_hbm.at[pl.ds(r0, rpt)], acc.at[pl.ds(r0, rpt)])

        @pl.when(sid == _NS - 1)
        def _():
            pltpu.sync_copy(zeros_hbm.at[pl.ds(rpt * _NS, Nn - rpt * _NS)],
                            acc.at[pl.ds(rpt * _NS, Nn - rpt * _NS)])

        pltpu.sync_copy(idx_hbm.at[pl.ds(j0, n_iter)], idx_v)
        pltpu.async_copy(vals_hbm.at[pl.ds(j0 * L, L)], v0, sv0)
        plsc.subcore_barrier()

        def slot(i, b, t):
            nb = 1 - b

            def wait_scatter_prev():
                pltpu.make_async_copy(v0, acc.at[idx_v.at[0]], ss).wait()

            def load_next():
                pltpu.async_copy(vals_hbm.at[pl.ds((j0 + i + 1) * L, L)],
                                 vb[nb], sv[nb])

            if b == 0:
                pl.when(t > 0)(wait_scatter_prev)
                load_next()
            else:
                wait_scatter_prev()
                pl.when(t < n_iter // 2 - 1)(load_next)
            pltpu.make_async_copy(vals_hbm.at[pl.ds(0, L)], vb[b], sv[b]).wait()
            pltpu.async_copy(vb[b], acc.at[idx_v.at[i]], ss, add=True)

        def outer(t, carry):
            slot(2 * t, 0, t)
            slot(2 * t + 1, 1, t)
            return carry

        lax.fori_loop(0, n_iter // 2, outer, 0)
        pltpu.make_async_copy(v0, acc.at[idx_v.at[0]], ss).wait()
        plsc.subcore_barrier()
        pltpu.sync_copy(acc.at[pl.ds(r0, rpt)],
                        out_hbm.at[pl.ds(cid * n_real + r0, rpt)])
        if tail:
            @pl.when(sid == _NS - 1)
            def _():
                pltpu.sync_copy(acc.at[pl.ds(rpt * _NS, tail)],
                                out_hbm.at[pl.ds(cid * n_real + rpt * _NS, tail)])

    return sk(vals, idx2d, zeros_nd)


def _ln(x, g, b):
    mu = jnp.mean(x, axis=-1, keepdims=True)
    xc = x - mu
    var = jnp.mean(xc * xc, axis=-1, keepdims=True)
    return xc * lax.rsqrt(var + _EPS) * g + b


def _full(shape):
    return pl.BlockSpec(shape, lambda i: (0, 0))


def _rows(blk, d):
    return pl.BlockSpec((blk, d), lambda i: (i, 0))


def _tc_vencode(vin, w1, b1, w2, b2, g, b, w1s, w1d, blk):
    Nn = vin.shape[0]

    def body(x_ref, w1_ref, b1_ref, w2_ref, b2_ref, g_ref, b_ref,
             ws_ref, wd_ref, v_ref, ps_ref, pd_ref):
        h = jnp.maximum(jnp.dot(x_ref[...], w1_ref[...],
                                preferred_element_type=jnp.float32) + b1_ref[...], 0.0)
        v = jnp.dot(h, w2_ref[...], preferred_element_type=jnp.float32) + b2_ref[...]
        v = _ln(v, g_ref[...], b_ref[...])
        v_ref[...] = v
        ps_ref[...] = jnp.dot(v, ws_ref[...], preferred_element_type=jnp.float32)
        pd_ref[...] = jnp.dot(v, wd_ref[...], preferred_element_type=jnp.float32)

    out = jax.ShapeDtypeStruct((Nn, H), jnp.float32)
    return pl.pallas_call(
        body,
        grid=(Nn // blk,),
        in_specs=[_rows(blk, vin.shape[1]), _full(w1.shape), _full((1, H)),
                  _full((H, H)), _full((1, H)), _full((1, H)), _full((1, H)),
                  _full((H, H)), _full((H, H))],
        out_specs=[_rows(blk, H)] * 3,
        out_shape=[out, out, out],
    )(vin, w1, b1, w2, b2, g, b, w1s, w1d)


def _tc_eencode(grow, gcol, w1, b1, w2, b2, g, b, blk):
    E = grow.shape[0]

    def body(gr_ref, gc_ref, w1_ref, b1_ref, w2_ref, b2_ref, g_ref, b_ref, o_ref):
        d = gc_ref[...] - gr_ref[...]
        w1 = w1_ref[...]
        sqe = jnp.sum(d[:, 0:3] * d[:, 0:3], axis=-1, keepdims=True)
        ne = jnp.where(sqe > 0, jnp.sqrt(jnp.where(sqe > 0, sqe, 1.0)), 0.0)
        sqm = jnp.sum(d[:, 3:6] * d[:, 3:6], axis=-1, keepdims=True)
        nm = jnp.where(sqm > 0, jnp.sqrt(jnp.where(sqm > 0, sqm, 1.0)), 0.0)
        h = jnp.broadcast_to(b1_ref[...], (blk, H))
        for k in range(3):
            h = h + d[:, k:k + 1] * w1[k:k + 1, :]
        h = h + ne * w1[3:4, :]
        for k in range(3):
            h = h + d[:, 3 + k:4 + k] * w1[4 + k:5 + k, :]
        h = h + nm * w1[7:8, :]
        h = jnp.maximum(h, 0.0)
        e = jnp.dot(h, w2_ref[...], preferred_element_type=jnp.float32) + b2_ref[...]
        o_ref[...] = _ln(e, g_ref[...], b_ref[...])

    return pl.pallas_call(
        body,
        grid=(E // blk,),
        in_specs=[_rows(blk, grow.shape[1]), _rows(blk, grow.shape[1]),
                  _full((8, H)), _full((1, H)), _full((H, H)), _full((1, H)),
                  _full((1, H)), _full((1, H))],
        out_specs=_rows(blk, H),
        out_shape=jax.ShapeDtypeStruct((E, H), jnp.float32),
    )(grow, gcol, w1, b1, w2, b2, g, b)


def _tc_edge(gs, gd, ea, w1e, b1, w2, b2, g, b, blk):
    E = ea.shape[0]

    def body(gs_ref, gd_ref, ea_ref, w1_ref, b1_ref, w2_ref, b2_ref,
             g_ref, b_ref, o_ref):
        ea_v = ea_ref[...]
        h = jnp.maximum(
            gs_ref[...] + gd_ref[...] + b1_ref[...]
            + jnp.dot(ea_v, w1_ref[...], preferred_element_type=jnp.float32), 0.0)
        e = jnp.dot(h, w2_ref[...], preferred_element_type=jnp.float32) \
            + b2_ref[...] + ea_v
        o_ref[...] = _ln(e, g_ref[...], b_ref[...])

    return pl.pallas_call(
        body,
        grid=(E // blk,),
        in_specs=[_rows(blk, H)] * 3
        + [_full((H, H)), _full((1, H)), _full((H, H)), _full((1, H)),
           _full((1, H)), _full((1, H))],
        out_specs=_rows(blk, H),
        out_shape=jax.ShapeDtypeStruct((E, H), jnp.float32),
    )(gs, gd, ea, w1e, b1, w2, b2, g, b)


def _tc_node(v, a0, a1, w1v, w1a, b1, w2, b2, g, b, w1s, w1d, blk):
    Nn = v.shape[0]

    def body(v_ref, a0_ref, a1_ref, w1v_ref, w1a_ref, b1_ref, w2_ref, b2_ref,
             g_ref, b_ref, ws_ref, wd_ref, vn_ref, ps_ref, pd_ref):
        v_v = v_ref[...]
        agg = a0_ref[...] + a1_ref[...]
        h = jnp.maximum(
            jnp.dot(v_v, w1v_ref[...], preferred_element_type=jnp.float32)
            + jnp.dot(agg, w1a_ref[...], preferred_element_type=jnp.float32)
            + b1_ref[...], 0.0)
        x = jnp.dot(h, w2_ref[...], preferred_element_type=jnp.float32) \
            + b2_ref[...] + v_v
        vn = _ln(x, g_ref[...], b_ref[...])
        vn_ref[...] = vn
        ps_ref[...] = jnp.dot(vn, ws_ref[...], preferred_element_type=jnp.float32)
        pd_ref[...] = jnp.dot(vn, wd_ref[...], preferred_element_type=jnp.float32)

    out = jax.ShapeDtypeStruct((Nn, H), jnp.float32)
    return pl.pallas_call(
        body,
        grid=(Nn // blk,),
        in_specs=[_rows(blk, H)] * 3
        + [_full((H, H)), _full((H, H)), _full((1, H)), _full((H, H)),
           _full((1, H)), _full((1, H)), _full((1, H)), _full((H, H)),
           _full((H, H))],
        out_specs=[_rows(blk, H)] * 3,
        out_shape=[out, out, out],
    )(v, a0, a1, w1v, w1a, b1, w2, b2, g, b, w1s, w1d)


def _tc_decode(v, w1, b1, w2p, b2p, blk):
    Nn = v.shape[0]

    def body(v_ref, w1_ref, b1_ref, w2_ref, b2_ref, o_ref):
        h = jnp.maximum(jnp.dot(v_ref[...], w1_ref[...],
                                preferred_element_type=jnp.float32) + b1_ref[...], 0.0)
        o_ref[...] = jnp.dot(h, w2_ref[...],
                             preferred_element_type=jnp.float32) + b2_ref[...]

    return pl.pallas_call(
        body,
        grid=(Nn // blk,),
        in_specs=[_rows(blk, H), _full((H, H)), _full((1, H)), _full((H, H)),
                  _full((1, H))],
        out_specs=_rows(blk, H),
        out_shape=jax.ShapeDtypeStruct((Nn, H), jnp.float32),
    )(v, w1, b1, w2p, b2p)


def kernel(world_coords, vertex_features, edge_index, static_nodes, mesh_coords,
           venc_w1, venc_b1, venc_w2, venc_b2,
           eenc_w1, eenc_b1, eenc_w2, eenc_b2,
           ln_g, ln_b,
           em_w1, em_b1, em_w2, em_b2, em_ln_g, em_ln_b,
           nm_w1, nm_b1, nm_w2, nm_b2, nm_ln_g, nm_ln_b,
           dec_w1, dec_b1, dec_w2, dec_b2):
    f32 = jnp.float32
    Nn = world_coords.shape[1]
    E = edge_index.shape[2]
    OUT = dec_w2.shape[1]
    nblk = 2000 if Nn % 2000 == 0 else Nn

    def r1(x):
        return x.reshape(1, -1)

    row0 = edge_index[0, 0]
    col0 = edge_index[0, 1]

    n_it = _cdiv(_cdiv(E, _L), _NW)
    n_it += n_it % 2
    C = n_it * _NW
    Epad = C * _L
    padg = jnp.zeros((Epad - E,), jnp.int32)
    rowg = jnp.concatenate([row0, padg]).reshape(C, _L)
    colg = jnp.concatenate([col0, padg]).reshape(C, _L)
    rows_idx = jnp.concatenate(
        [row0, jnp.full((Epad - E,), Nn, jnp.int32)]).reshape(C, _L)
    eblk = 2048 if Epad % 2048 == 0 else _L

    ct = jnp.concatenate(
        [world_coords[0], mesh_coords, jnp.zeros((Nn, H - 6), f32)], axis=1)
    grow, gcol = _sc_gather2(ct, ct, rowg, colg)
    ea = _tc_eencode(grow, gcol, eenc_w1, r1(eenc_b1), eenc_w2, r1(eenc_b2),
                     r1(ln_g), r1(ln_b), eblk)

    static_oh = jax.nn.one_hot(static_nodes, 2, dtype=f32)
    vin = jnp.concatenate(
        [static_oh, vertex_features[0], jnp.zeros((Nn, 4), f32)], axis=1)
    venc_w1p = jnp.concatenate([venc_w1, jnp.zeros((4, H), f32)], axis=0)
    em_w1s, em_w1d, em_w1e = em_w1[:H], em_w1[H:2 * H], em_w1[2 * H:]
    v, ps, pd = _tc_vencode(vin, venc_w1p, r1(venc_b1), venc_w2, r1(venc_b2),
                            r1(ln_g), r1(ln_b), em_w1s, em_w1d, nblk)

    zn = jnp.zeros((Nn + 16, H), f32)
    for _ in range(15):
        gs, gd = _sc_gather2(ps, pd, rowg, colg)
        ea = _tc_edge(gs, gd, ea, em_w1e, r1(em_b1), em_w2, r1(em_b2),
                      r1(em_ln_g), r1(em_ln_b), eblk)
        agg = _sc_scatter(ea, rows_idx, zn, Nn)
        v, ps, pd = _tc_node(v, agg[:Nn], agg[Nn:], nm_w1[:H], nm_w1[H:],
                             r1(nm_b1), nm_w2, r1(nm_b2),
                             r1(nm_ln_g), r1(nm_ln_b), em_w1s, em_w1d, nblk)

    dec_w2p = jnp.concatenate([dec_w2, jnp.zeros((H, H - OUT), f32)], axis=1)
    dec_b2p = jnp.concatenate([dec_b2, jnp.zeros((H - OUT,), f32)]).reshape(1, H)
    out = _tc_decode(v, dec_w1, r1(dec_b1), dec_w2p, dec_b2p, nblk)
    return out[:, :OUT].reshape(1, Nn, OUT)


# speedup vs baseline: 1.4661x; 1.2124x over previous
import functools

import jax
import jax.numpy as jnp
from jax import lax
from jax.experimental import pallas as pl
from jax.experimental.pallas import tpu as pltpu
from jax.experimental.pallas import tpu_sc as plsc

H = 128
_NC, _NS = 2, 16
_NW = _NC * _NS
_L = 128
_EPS = 1e-5


def _cdiv(a, b):
    return (a + b - 1) // b


def _sc_gather(table, idx2d):
    C, L = idx2d.shape
    D = table.shape[1]
    n_iter = _cdiv(C, _NW)
    mesh = plsc.VectorSubcoreMesh(core_axis_name="c", subcore_axis_name="s",
                                  num_cores=_NC, num_subcores=_NS)

    @functools.partial(
        pl.kernel,
        out_type=jax.ShapeDtypeStruct((C * L, D), jnp.float32),
        mesh=mesh,
        scratch_types=[
            pltpu.VMEM((L,), jnp.int32),
            pltpu.VMEM((L, D), jnp.float32),
            pltpu.SemaphoreType.DMA,
        ],
    )
    def gk(table_hbm, idx_hbm, out_hbm, idx_v, rows_v, sem):
        wid = lax.axis_index("s") * _NC + lax.axis_index("c")

        def body(i, carry):
            j = wid + i * _NW

            @pl.when(j < C)
            def _():
                pltpu.sync_copy(idx_hbm.at[j], idx_v)
                pltpu.async_copy(table_hbm.at[idx_v], rows_v, sem).wait()
                pltpu.sync_copy(rows_v, out_hbm.at[pl.ds(j * L, L)])

            return carry

        lax.fori_loop(0, n_iter, body, 0)

    return gk(table, idx2d)


def _sc_scatter(vals, idx2d, zeros_nd):
    C, L = idx2d.shape
    Nn, D = zeros_nd.shape
    rpt = (Nn // _NS) // 8 * 8
    tail = Nn - rpt * _NS
    per_sc = C // _NC
    n_iter = _cdiv(per_sc, _NS)
    mesh = plsc.VectorSubcoreMesh(core_axis_name="c", subcore_axis_name="s",
                                  num_cores=_NC, num_subcores=_NS)

    @functools.partial(
        pl.kernel,
        out_type=jax.ShapeDtypeStruct((_NC * Nn, D), jnp.float32),
        mesh=mesh,
        scratch_types=[
            pltpu.VMEM((L,), jnp.int32),
            pltpu.VMEM((L, D), jnp.float32),
            pltpu.VMEM_SHARED((Nn, D), jnp.float32),
            pltpu.SemaphoreType.DMA,
        ],
    )
    def sk(vals_hbm, idx_hbm, zeros_hbm, out_hbm, idx_v, vals_v, acc, sem):
        cid = lax.axis_index("c")
        sid = lax.axis_index("s")
        r0 = sid * rpt
        pltpu.sync_copy(zeros_hbm.at[pl.ds(r0, rpt)], acc.at[pl.ds(r0, rpt)])

        @pl.when(sid == _NS - 1)
        def _():
            pltpu.sync_copy(zeros_hbm.at[pl.ds(rpt * _NS, Nn - rpt * _NS)],
                            acc.at[pl.ds(rpt * _NS, Nn - rpt * _NS)])

        plsc.subcore_barrier()

        def body(i, carry):
            k = sid + i * _NS
            j = cid + k * _NC

            @pl.when(k < per_sc)
            def _():
                pltpu.sync_copy(idx_hbm.at[j], idx_v)
                pltpu.sync_copy(vals_hbm.at[pl.ds(j * L, L)], vals_v)
                pltpu.sync_copy(vals_v, acc.at[idx_v], add=True)

            return carry

        lax.fori_loop(0, n_iter, body, 0)
        plsc.subcore_barrier()
        pltpu.sync_copy(acc.at[pl.ds(r0, rpt)],
                        out_hbm.at[pl.ds(cid * Nn + r0, rpt)])
        if tail:
            @pl.when(sid == _NS - 1)
            def _():
                pltpu.sync_copy(acc.at[pl.ds(rpt * _NS, tail)],
                                out_hbm.at[pl.ds(cid * Nn + rpt * _NS, tail)])

    return sk(vals, idx2d, zeros_nd)


def _ln(x, g, b):
    mu = jnp.mean(x, axis=-1, keepdims=True)
    xc = x - mu
    var = jnp.mean(xc * xc, axis=-1, keepdims=True)
    return xc * lax.rsqrt(var + _EPS) * g + b


def _full(shape):
    return pl.BlockSpec(shape, lambda i: (0, 0))


def _rows(blk, d):
    return pl.BlockSpec((blk, d), lambda i: (i, 0))


def _tc_vencode(vin, w1, b1, w2, b2, g, b, w1s, w1d, blk):
    Nn = vin.shape[0]

    def body(x_ref, w1_ref, b1_ref, w2_ref, b2_ref, g_ref, b_ref,
             ws_ref, wd_ref, v_ref, ps_ref, pd_ref):
        h = jnp.maximum(jnp.dot(x_ref[...], w1_ref[...],
                                preferred_element_type=jnp.float32) + b1_ref[...], 0.0)
        v = jnp.dot(h, w2_ref[...], preferred_element_type=jnp.float32) + b2_ref[...]
        v = _ln(v, g_ref[...], b_ref[...])
        v_ref[...] = v
        ps_ref[...] = jnp.dot(v, ws_ref[...], preferred_element_type=jnp.float32)
        pd_ref[...] = jnp.dot(v, wd_ref[...], preferred_element_type=jnp.float32)

    out = jax.ShapeDtypeStruct((Nn, H), jnp.float32)
    return pl.pallas_call(
        body,
        grid=(Nn // blk,),
        in_specs=[_rows(blk, vin.shape[1]), _full(w1.shape), _full((1, H)),
                  _full((H, H)), _full((1, H)), _full((1, H)), _full((1, H)),
                  _full((H, H)), _full((H, H))],
        out_specs=[_rows(blk, H)] * 3,
        out_shape=[out, out, out],
    )(vin, w1, b1, w2, b2, g, b, w1s, w1d)


def _tc_eencode(grow, gcol, w1, b1, w2, b2, g, b, blk):
    E = grow.shape[0]

    def body(gr_ref, gc_ref, w1_ref, b1_ref, w2_ref, b2_ref, g_ref, b_ref, o_ref):
        d = gc_ref[...] - gr_ref[...]
        w1 = w1_ref[...]
        sqe = jnp.sum(d[:, 0:3] * d[:, 0:3], axis=-1, keepdims=True)
        ne = jnp.where(sqe > 0, jnp.sqrt(jnp.where(sqe > 0, sqe, 1.0)), 0.0)
        sqm = jnp.sum(d[:, 3:6] * d[:, 3:6], axis=-1, keepdims=True)
        nm = jnp.where(sqm > 0, jnp.sqrt(jnp.where(sqm > 0, sqm, 1.0)), 0.0)
        h = jnp.broadcast_to(b1_ref[...], (blk, H))
        for k in range(3):
            h = h + d[:, k:k + 1] * w1[k:k + 1, :]
        h = h + ne * w1[3:4, :]
        for k in range(3):
            h = h + d[:, 3 + k:4 + k] * w1[4 + k:5 + k, :]
        h = h + nm * w1[7:8, :]
        h = jnp.maximum(h, 0.0)
        e = jnp.dot(h, w2_ref[...], preferred_element_type=jnp.float32) + b2_ref[...]
        o_ref[...] = _ln(e, g_ref[...], b_ref[...])

    return pl.pallas_call(
        body,
        grid=(E // blk,),
        in_specs=[_rows(blk, grow.shape[1]), _rows(blk, grow.shape[1]),
                  _full((8, H)), _full((1, H)), _full((H, H)), _full((1, H)),
                  _full((1, H)), _full((1, H))],
        out_specs=_rows(blk, H),
        out_shape=jax.ShapeDtypeStruct((E, H), jnp.float32),
    )(grow, gcol, w1, b1, w2, b2, g, b)


def _tc_edge(gs, gd, ea, w1e, b1, w2, b2, g, b, blk):
    E = ea.shape[0]

    def body(gs_ref, gd_ref, ea_ref, w1_ref, b1_ref, w2_ref, b2_ref,
             g_ref, b_ref, o_ref):
        ea_v = ea_ref[...]
        h = jnp.maximum(
            gs_ref[...] + gd_ref[...] + b1_ref[...]
            + jnp.dot(ea_v, w1_ref[...], preferred_element_type=jnp.float32), 0.0)
        e = jnp.dot(h, w2_ref[...], preferred_element_type=jnp.float32) \
            + b2_ref[...] + ea_v
        o_ref[...] = _ln(e, g_ref[...], b_ref[...])

    return pl.pallas_call(
        body,
        grid=(E // blk,),
        in_specs=[_rows(blk, H)] * 3
        + [_full((H, H)), _full((1, H)), _full((H, H)), _full((1, H)),
           _full((1, H)), _full((1, H))],
        out_specs=_rows(blk, H),
        out_shape=jax.ShapeDtypeStruct((E, H), jnp.float32),
    )(gs, gd, ea, w1e, b1, w2, b2, g, b)


def _tc_node(v, a0, a1, w1v, w1a, b1, w2, b2, g, b, w1s, w1d, blk):
    Nn = v.shape[0]

    def body(v_ref, a0_ref, a1_ref, w1v_ref, w1a_ref, b1_ref, w2_ref, b2_ref,
             g_ref, b_ref, ws_ref, wd_ref, vn_ref, ps_ref, pd_ref):
        v_v = v_ref[...]
        agg = a0_ref[...] + a1_ref[...]
        h = jnp.maximum(
            jnp.dot(v_v, w1v_ref[...], preferred_element_type=jnp.float32)
            + jnp.dot(agg, w1a_ref[...], preferred_element_type=jnp.float32)
            + b1_ref[...], 0.0)
        x = jnp.dot(h, w2_ref[...], preferred_element_type=jnp.float32) \
            + b2_ref[...] + v_v
        vn = _ln(x, g_ref[...], b_ref[...])
        vn_ref[...] = vn
        ps_ref[...] = jnp.dot(vn, ws_ref[...], preferred_element_type=jnp.float32)
        pd_ref[...] = jnp.dot(vn, wd_ref[...], preferred_element_type=jnp.float32)

    out = jax.ShapeDtypeStruct((Nn, H), jnp.float32)
    return pl.pallas_call(
        body,
        grid=(Nn // blk,),
        in_specs=[_rows(blk, H)] * 3
        + [_full((H, H)), _full((H, H)), _full((1, H)), _full((H, H)),
           _full((1, H)), _full((1, H)), _full((1, H)), _full((H, H)),
           _full((H, H))],
        out_specs=[_rows(blk, H)] * 3,
        out_shape=[out, out, out],
    )(v, a0, a1, w1v, w1a, b1, w2, b2, g, b, w1s, w1d)


def _tc_decode(v, w1, b1, w2p, b2p, blk):
    Nn = v.shape[0]

    def body(v_ref, w1_ref, b1_ref, w2_ref, b2_ref, o_ref):
        h = jnp.maximum(jnp.dot(v_ref[...], w1_ref[...],
                                preferred_element_type=jnp.float32) + b1_ref[...], 0.0)
        o_ref[...] = jnp.dot(h, w2_ref[...],
                             preferred_element_type=jnp.float32) + b2_ref[...]

    return pl.pallas_call(
        body,
        grid=(Nn // blk,),
        in_specs=[_rows(blk, H), _full((H, H)), _full((1, H)), _full((H, H)),
                  _full((1, H))],
        out_specs=_rows(blk, H),
        out_shape=jax.ShapeDtypeStruct((Nn, H), jnp.float32),
    )(v, w1, b1, w2p, b2p)


def kernel(world_coords, vertex_features, edge_index, static_nodes, mesh_coords,
           venc_w1, venc_b1, venc_w2, venc_b2,
           eenc_w1, eenc_b1, eenc_w2, eenc_b2,
           ln_g, ln_b,
           em_w1, em_b1, em_w2, em_b2, em_ln_g, em_ln_b,
           nm_w1, nm_b1, nm_w2, nm_b2, nm_ln_g, nm_ln_b,
           dec_w1, dec_b1, dec_w2, dec_b2):
    f32 = jnp.float32
    Nn = world_coords.shape[1]
    E = edge_index.shape[2]
    OUT = dec_w2.shape[1]
    nblk = 2000 if Nn % 2000 == 0 else Nn

    def r1(x):
        return x.reshape(1, -1)

    row = edge_index[0, 0].reshape(-1, _L)
    col = edge_index[0, 1].reshape(-1, _L)
    eblk = 2000 if E % 2000 == 0 else E

    ct = jnp.concatenate(
        [world_coords[0], mesh_coords, jnp.zeros((Nn, H - 6), f32)], axis=1)
    grow = _sc_gather(ct, row)
    gcol = _sc_gather(ct, col)
    ea = _tc_eencode(grow, gcol, eenc_w1, r1(eenc_b1), eenc_w2, r1(eenc_b2),
                     r1(ln_g), r1(ln_b), eblk)

    static_oh = jax.nn.one_hot(static_nodes, 2, dtype=f32)
    vin = jnp.concatenate(
        [static_oh, vertex_features[0], jnp.zeros((Nn, 4), f32)], axis=1)
    venc_w1p = jnp.concatenate([venc_w1, jnp.zeros((4, H), f32)], axis=0)
    em_w1s, em_w1d, em_w1e = em_w1[:H], em_w1[H:2 * H], em_w1[2 * H:]
    v, ps, pd = _tc_vencode(vin, venc_w1p, r1(venc_b1), venc_w2, r1(venc_b2),
                            r1(ln_g), r1(ln_b), em_w1s, em_w1d, nblk)

    zn = jnp.zeros((Nn, H), f32)
    for _ in range(15):
        gs = _sc_gather(ps, row)
        gd = _sc_gather(pd, col)
        ea = _tc_edge(gs, gd, ea, em_w1e, r1(em_b1), em_w2, r1(em_b2),
                      r1(em_ln_g), r1(em_ln_b), eblk)
        agg = _sc_scatter(ea, row, zn)
        v, ps, pd = _tc_node(v, agg[:Nn], agg[Nn:], nm_w1[:H], nm_w1[H:],
                             r1(nm_b1), nm_w2, r1(nm_b2),
                             r1(nm_ln_g), r1(nm_ln_b), em_w1s, em_w1d, nblk)

    dec_w2p = jnp.concatenate([dec_w2, jnp.zeros((H, H - OUT), f32)], axis=1)
    dec_b2p = jnp.concatenate([dec_b2, jnp.zeros((H - OUT,), f32)]).reshape(1, H)
    out = _tc_decode(v, dec_w1, r1(dec_b1), dec_w2p, dec_b2p, nblk)
    return out[:, :OUT].reshape(1, Nn, OUT)


# speedup vs baseline: 1.6378x; 1.1171x over previous
import functools

import jax
import jax.numpy as jnp
from jax import lax
from jax.experimental import pallas as pl
from jax.experimental.pallas import tpu as pltpu
from jax.experimental.pallas import tpu_sc as plsc

H = 128
_NC, _NS = 2, 16
_NW = _NC * _NS
_L = 128
_EPS = 1e-5


def _cdiv(a, b):
    return (a + b - 1) // b


def _sc_gather(table, idx2d):
    C, L = idx2d.shape
    D = table.shape[1]
    n_iter = _cdiv(C, _NW)
    mesh = plsc.VectorSubcoreMesh(core_axis_name="c", subcore_axis_name="s",
                                  num_cores=_NC, num_subcores=_NS)

    @functools.partial(
        pl.kernel,
        out_type=jax.ShapeDtypeStruct((C * L, D), jnp.float32),
        mesh=mesh,
        scratch_types=[
            pltpu.VMEM((n_iter, L), jnp.int32),
            pltpu.VMEM((L, D), jnp.float32),
            pltpu.SemaphoreType.DMA,
            pltpu.SemaphoreType.DMA,
        ],
    )
    def gk(table_hbm, idx_hbm, out_hbm, idx_v, rows_v, sem, isem):
        wid = lax.axis_index("s") * _NC + lax.axis_index("c")

        def pre(i, carry):
            @pl.when(wid + i * _NW < C)
            def _():
                pltpu.async_copy(idx_hbm.at[wid + i * _NW], idx_v.at[i], isem)
            return carry

        lax.fori_loop(0, n_iter, pre, 0)

        def drain(i, carry):
            @pl.when(wid + i * _NW < C)
            def _():
                pltpu.make_async_copy(idx_hbm.at[0], idx_v.at[0], isem).wait()
            return carry

        lax.fori_loop(0, n_iter, drain, 0)

        def body(i, carry):
            j = wid + i * _NW

            @pl.when(j < C)
            def _():
                pltpu.async_copy(table_hbm.at[idx_v.at[i]], rows_v, sem).wait()
                pltpu.sync_copy(rows_v, out_hbm.at[pl.ds(j * L, L)])

            return carry

        lax.fori_loop(0, n_iter, body, 0)

    return gk(table, idx2d)


def _sc_scatter(vals, idx2d, zeros_nd):
    C, L = idx2d.shape
    Nn, D = zeros_nd.shape
    rpt = (Nn // _NS) // 8 * 8
    tail = Nn - rpt * _NS
    per_sc = C // _NC
    n_iter = _cdiv(per_sc, _NS)
    mesh = plsc.VectorSubcoreMesh(core_axis_name="c", subcore_axis_name="s",
                                  num_cores=_NC, num_subcores=_NS)

    @functools.partial(
        pl.kernel,
        out_type=jax.ShapeDtypeStruct((_NC * Nn, D), jnp.float32),
        mesh=mesh,
        scratch_types=[
            pltpu.VMEM((n_iter, L), jnp.int32),
            pltpu.VMEM((L, D), jnp.float32),
            pltpu.VMEM_SHARED((Nn, D), jnp.float32),
            pltpu.SemaphoreType.DMA,
            pltpu.SemaphoreType.DMA,
        ],
    )
    def sk(vals_hbm, idx_hbm, zeros_hbm, out_hbm, idx_v, vals_v, acc, sem, isem):
        cid = lax.axis_index("c")
        sid = lax.axis_index("s")
        r0 = sid * rpt
        pltpu.sync_copy(zeros_hbm.at[pl.ds(r0, rpt)], acc.at[pl.ds(r0, rpt)])

        @pl.when(sid == _NS - 1)
        def _():
            pltpu.sync_copy(zeros_hbm.at[pl.ds(rpt * _NS, Nn - rpt * _NS)],
                            acc.at[pl.ds(rpt * _NS, Nn - rpt * _NS)])

        def pre(i, carry):
            @pl.when(sid + i * _NS < per_sc)
            def _():
                pltpu.async_copy(idx_hbm.at[cid + (sid + i * _NS) * _NC],
                                 idx_v.at[i], isem)
            return carry

        lax.fori_loop(0, n_iter, pre, 0)

        def drain(i, carry):
            @pl.when(sid + i * _NS < per_sc)
            def _():
                pltpu.make_async_copy(idx_hbm.at[0], idx_v.at[0], isem).wait()
            return carry

        lax.fori_loop(0, n_iter, drain, 0)
        plsc.subcore_barrier()

        def body(i, carry):
            k = sid + i * _NS
            j = cid + k * _NC

            @pl.when(k < per_sc)
            def _():
                pltpu.sync_copy(vals_hbm.at[pl.ds(j * L, L)], vals_v)
                pltpu.sync_copy(vals_v, acc.at[idx_v.at[i]], add=True)

            return carry

        lax.fori_loop(0, n_iter, body, 0)
        plsc.subcore_barrier()
        pltpu.sync_copy(acc.at[pl.ds(r0, rpt)],
                        out_hbm.at[pl.ds(cid * Nn + r0, rpt)])
        if tail:
            @pl.when(sid == _NS - 1)
            def _():
                pltpu.sync_copy(acc.at[pl.ds(rpt * _NS, tail)],
                                out_hbm.at[pl.ds(cid * Nn + rpt * _NS, tail)])

    return sk(vals, idx2d, zeros_nd)


def _ln(x, g, b):
    mu = jnp.mean(x, axis=-1, keepdims=True)
    xc = x - mu
    var = jnp.mean(xc * xc, axis=-1, keepdims=True)
    return xc * lax.rsqrt(var + _EPS) * g + b


def _full(shape):
    return pl.BlockSpec(shape, lambda i: (0, 0))


def _rows(blk, d):
    return pl.BlockSpec((blk, d), lambda i: (i, 0))


def _tc_vencode(vin, w1, b1, w2, b2, g, b, w1s, w1d, blk):
    Nn = vin.shape[0]

    def body(x_ref, w1_ref, b1_ref, w2_ref, b2_ref, g_ref, b_ref,
             ws_ref, wd_ref, v_ref, ps_ref, pd_ref):
        h = jnp.maximum(jnp.dot(x_ref[...], w1_ref[...],
                                preferred_element_type=jnp.float32) + b1_ref[...], 0.0)
        v = jnp.dot(h, w2_ref[...], preferred_element_type=jnp.float32) + b2_ref[...]
        v = _ln(v, g_ref[...], b_ref[...])
        v_ref[...] = v
        ps_ref[...] = jnp.dot(v, ws_ref[...], preferred_element_type=jnp.float32)
        pd_ref[...] = jnp.dot(v, wd_ref[...], preferred_element_type=jnp.float32)

    out = jax.ShapeDtypeStruct((Nn, H), jnp.float32)
    return pl.pallas_call(
        body,
        grid=(Nn // blk,),
        in_specs=[_rows(blk, vin.shape[1]), _full(w1.shape), _full((1, H)),
                  _full((H, H)), _full((1, H)), _full((1, H)), _full((1, H)),
                  _full((H, H)), _full((H, H))],
        out_specs=[_rows(blk, H)] * 3,
        out_shape=[out, out, out],
    )(vin, w1, b1, w2, b2, g, b, w1s, w1d)


def _tc_eencode(grow, gcol, w1, b1, w2, b2, g, b, blk):
    E = grow.shape[0]

    def body(gr_ref, gc_ref, w1_ref, b1_ref, w2_ref, b2_ref, g_ref, b_ref, o_ref):
        d = gc_ref[...] - gr_ref[...]
        w1 = w1_ref[...]
        sqe = jnp.sum(d[:, 0:3] * d[:, 0:3], axis=-1, keepdims=True)
        ne = jnp.where(sqe > 0, jnp.sqrt(jnp.where(sqe > 0, sqe, 1.0)), 0.0)
        sqm = jnp.sum(d[:, 3:6] * d[:, 3:6], axis=-1, keepdims=True)
        nm = jnp.where(sqm > 0, jnp.sqrt(jnp.where(sqm > 0, sqm, 1.0)), 0.0)
        h = jnp.broadcast_to(b1_ref[...], (blk, H))
        for k in range(3):
            h = h + d[:, k:k + 1] * w1[k:k + 1, :]
        h = h + ne * w1[3:4, :]
        for k in range(3):
            h = h + d[:, 3 + k:4 + k] * w1[4 + k:5 + k, :]
        h = h + nm * w1[7:8, :]
        h = jnp.maximum(h, 0.0)
        e = jnp.dot(h, w2_ref[...], preferred_element_type=jnp.float32) + b2_ref[...]
        o_ref[...] = _ln(e, g_ref[...], b_ref[...])

    return pl.pallas_call(
        body,
        grid=(E // blk,),
        in_specs=[_rows(blk, grow.shape[1]), _rows(blk, grow.shape[1]),
                  _full((8, H)), _full((1, H)), _full((H, H)), _full((1, H)),
                  _full((1, H)), _full((1, H))],
        out_specs=_rows(blk, H),
        out_shape=jax.ShapeDtypeStruct((E, H), jnp.float32),
    )(grow, gcol, w1, b1, w2, b2, g, b)


def _tc_edge(gs, gd, ea, w1e, b1, w2, b2, g, b, blk):
    E = ea.shape[0]

    def body(gs_ref, gd_ref, ea_ref, w1_ref, b1_ref, w2_ref, b2_ref,
             g_ref, b_ref, o_ref):
        ea_v = ea_ref[...]
        h = jnp.maximum(
            gs_ref[...] + gd_ref[...] + b1_ref[...]
            + jnp.dot(ea_v, w1_ref[...], preferred_element_type=jnp.float32), 0.0)
        e = jnp.dot(h, w2_ref[...], preferred_element_type=jnp.float32) \
            + b2_ref[...] + ea_v
        o_ref[...] = _ln(e, g_ref[...], b_ref[...])

    return pl.pallas_call(
        body,
        grid=(E // blk,),
        in_specs=[_rows(blk, H)] * 3
        + [_full((H, H)), _full((1, H)), _full((H, H)), _full((1, H)),
           _full((1, H)), _full((1, H))],
        out_specs=_rows(blk, H),
        out_shape=jax.ShapeDtypeStruct((E, H), jnp.float32),
    )(gs, gd, ea, w1e, b1, w2, b2, g, b)


def _tc_node(v, a0, a1, w1v, w1a, b1, w2, b2, g, b, w1s, w1d, blk):
    Nn = v.shape[0]

    def body(v_ref, a0_ref, a1_ref, w1v_ref, w1a_ref, b1_ref, w2_ref, b2_ref,
             g_ref, b_ref, ws_ref, wd_ref, vn_ref, ps_ref, pd_ref):
        v_v = v_ref[...]
        agg = a0_ref[...] + a1_ref[...]
        h = jnp.maximum(
            jnp.dot(v_v, w1v_ref[...], preferred_element_type=jnp.float32)
            + jnp.dot(agg, w1a_ref[...], preferred_element_type=jnp.float32)
            + b1_ref[...], 0.0)
        x = jnp.dot(h, w2_ref[...], preferred_element_type=jnp.float32) \
            + b2_ref[...] + v_v
        vn = _ln(x, g_ref[...], b_ref[...])
        vn_ref[...] = vn
        ps_ref[...] = jnp.dot(vn, ws_ref[...], preferred_element_type=jnp.float32)
        pd_ref[...] = jnp.dot(vn, wd_ref[...], preferred_element_type=jnp.float32)

    out = jax.ShapeDtypeStruct((Nn, H), jnp.float32)
    return pl.pallas_call(
        body,
        grid=(Nn // blk,),
        in_specs=[_rows(blk, H)] * 3
        + [_full((H, H)), _full((H, H)), _full((1, H)), _full((H, H)),
           _full((1, H)), _full((1, H)), _full((1, H)), _full((H, H)),
           _full((H, H))],
        out_specs=[_rows(blk, H)] * 3,
        out_shape=[out, out, out],
    )(v, a0, a1, w1v, w1a, b1, w2, b2, g, b, w1s, w1d)


def _tc_decode(v, w1, b1, w2p, b2p, blk):
    Nn = v.shape[0]

    def body(v_ref, w1_ref, b1_ref, w2_ref, b2_ref, o_ref):
        h = jnp.maximum(jnp.dot(v_ref[...], w1_ref[...],
                                preferred_element_type=jnp.float32) + b1_ref[...], 0.0)
        o_ref[...] = jnp.dot(h, w2_ref[...],
                             preferred_element_type=jnp.float32) + b2_ref[...]

    return pl.pallas_call(
        body,
        grid=(Nn // blk,),
        in_specs=[_rows(blk, H), _full((H, H)), _full((1, H)), _full((H, H)),
                  _full((1, H))],
        out_specs=_rows(blk, H),
        out_shape=jax.ShapeDtypeStruct((Nn, H), jnp.float32),
    )(v, w1, b1, w2p, b2p)


def kernel(world_coords, vertex_features, edge_index, static_nodes, mesh_coords,
           venc_w1, venc_b1, venc_w2, venc_b2,
           eenc_w1, eenc_b1, eenc_w2, eenc_b2,
           ln_g, ln_b,
           em_w1, em_b1, em_w2, em_b2, em_ln_g, em_ln_b,
           nm_w1, nm_b1, nm_w2, nm_b2, nm_ln_g, nm_ln_b,
           dec_w1, dec_b1, dec_w2, dec_b2):
    f32 = jnp.float32
    Nn = world_coords.shape[1]
    E = edge_index.shape[2]
    OUT = dec_w2.shape[1]
    nblk = 2000 if Nn % 2000 == 0 else Nn

    def r1(x):
        return x.reshape(1, -1)

    row = edge_index[0, 0].reshape(-1, _L)
    col = edge_index[0, 1].reshape(-1, _L)
    eblk = 2000 if E % 2000 == 0 else E

    ct = jnp.concatenate(
        [world_coords[0], mesh_coords, jnp.zeros((Nn, H - 6), f32)], axis=1)
    grow = _sc_gather(ct, row)
    gcol = _sc_gather(ct, col)
    ea = _tc_eencode(grow, gcol, eenc_w1, r1(eenc_b1), eenc_w2, r1(eenc_b2),
                     r1(ln_g), r1(ln_b), eblk)

    static_oh = jax.nn.one_hot(static_nodes, 2, dtype=f32)
    vin = jnp.concatenate(
        [static_oh, vertex_features[0], jnp.zeros((Nn, 4), f32)], axis=1)
    venc_w1p = jnp.concatenate([venc_w1, jnp.zeros((4, H), f32)], axis=0)
    em_w1s, em_w1d, em_w1e = em_w1[:H], em_w1[H:2 * H], em_w1[2 * H:]
    v, ps, pd = _tc_vencode(vin, venc_w1p, r1(venc_b1), venc_w2, r1(venc_b2),
                            r1(ln_g), r1(ln_b), em_w1s, em_w1d, nblk)

    zn = jnp.zeros((Nn, H), f32)
    for _ in range(15):
        gs = _sc_gather(ps, row)
        gd = _sc_gather(pd, col)
        ea = _tc_edge(gs, gd, ea, em_w1e, r1(em_b1), em_w2, r1(em_b2),
                      r1(em_ln_g), r1(em_ln_b), eblk)
        agg = _sc_scatter(ea, row, zn)
        v, ps, pd = _tc_node(v, agg[:Nn], agg[Nn:], nm_w1[:H], nm_w1[H:],
                             r1(nm_b1), nm_w2, r1(nm_b2),
                             r1(nm_ln_g), r1(nm_ln_b), em_w1s, em_w1d, nblk)

    dec_w2p = jnp.concatenate([dec_w2, jnp.zeros((H, H - OUT), f32)], axis=1)
    dec_b2p = jnp.concatenate([dec_b2, jnp.zeros((H - OUT,), f32)]).reshape(1, H)
    out = _tc_decode(v, dec_w1, r1(dec_b1), dec_w2p, dec_b2p, nblk)
    return out[:, :OUT].reshape(1, Nn, OUT)
